# baseline (speedup 1.0000x reference)
import functools

import jax
import jax.numpy as jnp
from jax import lax
from jax.experimental import pallas as pl
from jax.experimental.pallas import tpu as pltpu
from jax.experimental.pallas import tpu_sc as plsc

_EMBED = 16
_SEQ = 200
_NS = 16
_B_PAD = 256
_B_PER_W = _B_PAD // _NS


def _tile_body(table_hbm, idx_hbm, out_hbm, idx_v, rows_v, part_v, all_v,
               shared, sem):
    wid = lax.axis_index("s")
    base = wid * _B_PER_W
    pltpu.sync_copy(idx_hbm.at[pl.ds(base, _B_PER_W)], idx_v)
    pltpu.async_copy(table_hbm.at[idx_v], rows_v, sem).wait()
    m = rows_v[0, :]
    for i in range(1, _B_PER_W):
        m = jnp.maximum(m, rows_v[i, :])
    part_v[...] = m
    pltpu.sync_copy(part_v, shared.at[wid])
    plsc.subcore_barrier()

    @pl.when(wid == 0)
    def _():
        pltpu.sync_copy(shared, all_v)
        f = all_v[0, :]
        for i in range(1, _NS):
            f = jnp.maximum(f, all_v[i, :])
        part_v[...] = f
        pltpu.sync_copy(part_v, out_hbm)


@jax.jit
def kernel(words, table):
    idx = words.astype(jnp.int32)
    pad = jnp.broadcast_to(idx[0], (_B_PAD - _SEQ,))
    idx_p = jnp.concatenate([idx, pad])

    mesh = plsc.VectorSubcoreMesh(
        core_axis_name="c", subcore_axis_name="s", num_cores=1)
    run = pl.kernel(
        _tile_body,
        out_type=jax.ShapeDtypeStruct((_EMBED,), jnp.float32),
        mesh=mesh,
        scratch_types=[
            pltpu.VMEM((_B_PER_W,), jnp.int32),
            pltpu.VMEM((_B_PER_W, _EMBED), jnp.float32),
            pltpu.VMEM((_EMBED,), jnp.float32),
            pltpu.VMEM((_NS, _EMBED), jnp.float32),
            pltpu.VMEM_SHARED((_NS, _EMBED), jnp.float32),
            pltpu.SemaphoreType.DMA,
        ],
        compiler_params=pltpu.CompilerParams(use_tc_tiling_on_sc=False),
    )
    return run(table, idx_p)
